# baseline (speedup 1.0000x reference)
import functools

import jax
import jax.numpy as jnp
from jax import lax
from jax.experimental import pallas as pl
from jax.experimental.pallas import tpu as pltpu
from jax.experimental.pallas import tpu_sc as plsc

VOCAB = 100000
D = 64
B = 1024
CTX = 20

NC = 2
NS = 16
NW = NC * NS
BPW = B // NW
IPW = BPW * CTX
ICHUNK = 128
NGATHER = IPW // ICHUNK

TV = 1024
NV = (VOCAB + TV - 1) // TV
TAIL = VOCAB - (NV - 1) * TV


@functools.partial(
    pl.kernel,
    out_type=[
        jax.ShapeDtypeStruct((B, D), jnp.float32),
        jax.ShapeDtypeStruct((B, D), jnp.float32),
        jax.ShapeDtypeStruct((B,), jnp.float32),
    ],
    mesh=plsc.VectorSubcoreMesh(core_axis_name="c", subcore_axis_name="s"),
    compiler_params=pltpu.CompilerParams(use_tc_tiling_on_sc=False),
    scratch_types=[
        pltpu.VMEM((NGATHER, ICHUNK), jnp.int32),
        pltpu.VMEM((BPW,), jnp.int32),
        pltpu.VMEM((IPW, D), jnp.float32),
        pltpu.VMEM((BPW, D), jnp.float32),
        pltpu.VMEM((BPW,), jnp.float32),
        pltpu.VMEM((BPW, D), jnp.float32),
        pltpu.SemaphoreType.DMA,
    ],
)
def _sc_gather_pool(ctx_hbm, cen_hbm, emb_hbm, w_hbm, b_hbm,
                    ectx_out, wc_out, bc_out,
                    idx_v, cidx_v, rows_v, wc_v, bc_v, acc_v, sem):
    wid = lax.axis_index("s") * NC + lax.axis_index("c")

    pltpu.sync_copy(ctx_hbm.at[wid], idx_v)
    pltpu.sync_copy(cen_hbm.at[pl.ds(wid * BPW, BPW)], cidx_v)

    copies = [
        pltpu.async_copy(emb_hbm.at[idx_v.at[k]],
                         rows_v.at[pl.ds(k * ICHUNK, ICHUNK)], sem)
        for k in range(NGATHER)
    ]
    copies.append(pltpu.async_copy(w_hbm.at[cidx_v], wc_v, sem))
    copies.append(pltpu.async_copy(b_hbm.at[cidx_v], bc_v, sem))
    for c in copies:
        c.wait()

    def pool_row(r, carry):
        base = r * CTX
        for c in range(D // 16):
            acc = rows_v[base, pl.ds(c * 16, 16)]
            for t in range(1, CTX):
                acc = acc + rows_v[base + t, pl.ds(c * 16, 16)]
            acc_v[r, pl.ds(c * 16, 16)] = acc * (1.0 / CTX)
        return carry

    lax.fori_loop(0, BPW, pool_row, 0)

    pltpu.sync_copy(acc_v, ectx_out.at[pl.ds(wid * BPW, BPW)])
    pltpu.sync_copy(wc_v, wc_out.at[pl.ds(wid * BPW, BPW)])
    pltpu.sync_copy(bc_v, bc_out.at[pl.ds(wid * BPW, BPW)])


def _tc_body(e_ref, wc_ref, bc_ref, w_ref, b_ref, out_ref, loss_ref, s_ref):
    j = pl.program_id(0)

    @pl.when(j == 0)
    def _init():
        s_ref[...] = jnp.zeros_like(s_ref)

    logits = lax.dot_general(
        e_ref[...], w_ref[...], (((1,), (1,)), ((), ())),
        preferred_element_type=jnp.float32) + b_ref[...]
    out_ref[...] = logits

    @pl.when(j < NV - 1)
    def _acc():
        s_ref[...] += jnp.sum(jnp.exp(logits), axis=1, keepdims=True)

    @pl.when(j == NV - 1)
    def _tail():
        col = lax.broadcasted_iota(jnp.int32, logits.shape, 1)
        ex = jnp.where(col < TAIL, jnp.exp(logits), 0.0)
        s = s_ref[...] + jnp.sum(ex, axis=1, keepdims=True)
        center_logit = (jnp.sum(e_ref[...] * wc_ref[...], axis=1, keepdims=True)
                        + bc_ref[...])
        loss_ref[0, 0] = jnp.mean(jnp.log(s) - center_logit)


_tc_project = pl.pallas_call(
    _tc_body,
    grid=(NV,),
    in_specs=[
        pl.BlockSpec((B, D), lambda j: (0, 0)),
        pl.BlockSpec((B, D), lambda j: (0, 0)),
        pl.BlockSpec((B, 1), lambda j: (0, 0)),
        pl.BlockSpec((TV, D), lambda j: (j, 0)),
        pl.BlockSpec((1, TV), lambda j: (0, j)),
    ],
    out_specs=[
        pl.BlockSpec((B, TV), lambda j: (0, j)),
        pl.BlockSpec(memory_space=pltpu.SMEM),
    ],
    out_shape=[
        jax.ShapeDtypeStruct((B, VOCAB), jnp.float32),
        jax.ShapeDtypeStruct((1, 1), jnp.float32),
    ],
    scratch_shapes=[pltpu.VMEM((B, 1), jnp.float32)],
    compiler_params=pltpu.CompilerParams(dimension_semantics=("arbitrary",)),
)


def kernel(centers, contexts, emb, W, b):
    centers = centers.astype(jnp.int32)
    ctx3d = contexts.astype(jnp.int32).reshape(NW, NGATHER, ICHUNK)
    e_ctx, wc, bc = _sc_gather_pool(ctx3d, centers, emb, W, b)
    logits, loss2d = _tc_project(e_ctx, wc, bc.reshape(B, 1), W,
                                 b.reshape(1, VOCAB))
    return logits, loss2d[0, 0]


# speedup vs baseline: 1.8579x; 1.8579x over previous
import functools

import jax
import jax.numpy as jnp
from jax import lax
from jax.experimental import pallas as pl
from jax.experimental.pallas import tpu as pltpu
from jax.experimental.pallas import tpu_sc as plsc

VOCAB = 100000
D = 64
B = 1024
CTX = 20

NC = 2
NS = 16
NW = NC * NS
BPW = B // NW
IPW = BPW * CTX
ICHUNK = 128
NGATHER = IPW // ICHUNK

TV = 1024
NV = (VOCAB + TV - 1) // TV
TAIL = VOCAB - (NV - 1) * TV


@functools.partial(
    pl.kernel,
    out_type=[
        jax.ShapeDtypeStruct((B, D), jnp.float32),
        jax.ShapeDtypeStruct((B, D), jnp.float32),
        jax.ShapeDtypeStruct((B,), jnp.float32),
    ],
    mesh=plsc.VectorSubcoreMesh(core_axis_name="c", subcore_axis_name="s"),
    compiler_params=pltpu.CompilerParams(use_tc_tiling_on_sc=False),
    scratch_types=[
        pltpu.VMEM((NGATHER, ICHUNK), jnp.int32),
        pltpu.VMEM((BPW,), jnp.int32),
        pltpu.VMEM((IPW, D), jnp.float32),
        pltpu.VMEM((BPW, D), jnp.float32),
        pltpu.VMEM((BPW,), jnp.float32),
        pltpu.VMEM((BPW, D), jnp.float32),
        pltpu.SemaphoreType.DMA,
    ],
)
def _sc_gather_pool(ctx_hbm, cen_hbm, emb_hbm, w_hbm, b_hbm,
                    ectx_out, wc_out, bc_out,
                    idx_v, cidx_v, rows_v, wc_v, bc_v, acc_v, sem):
    wid = lax.axis_index("s") * NC + lax.axis_index("c")

    pltpu.sync_copy(ctx_hbm.at[wid], idx_v)
    pltpu.sync_copy(cen_hbm.at[pl.ds(wid * BPW, BPW)], cidx_v)

    copies = [
        pltpu.async_copy(emb_hbm.at[idx_v.at[k]],
                         rows_v.at[pl.ds(k * ICHUNK, ICHUNK)], sem)
        for k in range(NGATHER)
    ]
    copies.append(pltpu.async_copy(w_hbm.at[cidx_v], wc_v, sem))
    copies.append(pltpu.async_copy(b_hbm.at[cidx_v], bc_v, sem))
    for c in copies:
        c.wait()

    def pool_row(r, carry):
        base = r * CTX
        for c in range(D // 16):
            acc = rows_v[base, pl.ds(c * 16, 16)]
            for t in range(1, CTX):
                acc = acc + rows_v[base + t, pl.ds(c * 16, 16)]
            acc_v[r, pl.ds(c * 16, 16)] = acc * (1.0 / CTX)
        return carry

    lax.fori_loop(0, BPW, pool_row, 0)

    pltpu.sync_copy(acc_v, ectx_out.at[pl.ds(wid * BPW, BPW)])
    pltpu.sync_copy(wc_v, wc_out.at[pl.ds(wid * BPW, BPW)])
    pltpu.sync_copy(bc_v, bc_out.at[pl.ds(wid * BPW, BPW)])


def _tc_body(eT_ref, wcT_ref, bcT_ref, wT_ref, b_ref, out_ref, loss_ref, s_ref):
    j = pl.program_id(0)

    @pl.when(j == 0)
    def _init():
        s_ref[...] = jnp.zeros_like(s_ref)

    logitsT = lax.dot_general(
        wT_ref[...], eT_ref[...], (((0,), (0,)), ((), ())),
        preferred_element_type=jnp.float32) + b_ref[...]
    out_ref[...] = logitsT

    @pl.when(j < NV - 1)
    def _acc():
        s_ref[...] += jnp.sum(jnp.exp(logitsT), axis=0, keepdims=True)

    @pl.when(j == NV - 1)
    def _tail():
        row = lax.broadcasted_iota(jnp.int32, logitsT.shape, 0)
        ex = jnp.where(row < TAIL, jnp.exp(logitsT), 0.0)
        s = s_ref[...] + jnp.sum(ex, axis=0, keepdims=True)
        center_logit = (jnp.sum(eT_ref[...] * wcT_ref[...], axis=0,
                                keepdims=True) + bcT_ref[...])
        loss_ref[0, 0] = jnp.mean(jnp.log(s) - center_logit)


_tc_project = pl.pallas_call(
    _tc_body,
    grid=(NV,),
    in_specs=[
        pl.BlockSpec((D, B), lambda j: (0, 0)),
        pl.BlockSpec((D, B), lambda j: (0, 0)),
        pl.BlockSpec((1, B), lambda j: (0, 0)),
        pl.BlockSpec((D, TV), lambda j: (0, j)),
        pl.BlockSpec((TV, 1), lambda j: (j, 0)),
    ],
    out_specs=[
        pl.BlockSpec((TV, B), lambda j: (j, 0)),
        pl.BlockSpec(memory_space=pltpu.SMEM),
    ],
    out_shape=[
        jax.ShapeDtypeStruct((VOCAB, B), jnp.float32),
        jax.ShapeDtypeStruct((1, 1), jnp.float32),
    ],
    scratch_shapes=[pltpu.VMEM((1, B), jnp.float32)],
    compiler_params=pltpu.CompilerParams(dimension_semantics=("arbitrary",)),
)


def kernel(centers, contexts, emb, W, b):
    centers = centers.astype(jnp.int32)
    ctx3d = contexts.astype(jnp.int32).reshape(NW, NGATHER, ICHUNK)
    e_ctx, wc, bc = _sc_gather_pool(ctx3d, centers, emb, W, b)
    logitsT, loss2d = _tc_project(e_ctx.T, wc.T, bc.reshape(1, B), W.T,
                                  b[:, None])
    return logitsT.T, loss2d[0, 0]


# speedup vs baseline: 2.1450x; 1.1545x over previous
import functools

import jax
import jax.numpy as jnp
from jax import lax
from jax.experimental import pallas as pl
from jax.experimental.pallas import tpu as pltpu
from jax.experimental.pallas import tpu_sc as plsc

VOCAB = 100000
D = 64
B = 1024
CTX = 20

NC = 2
NS = 16
NW = NC * NS
BPW = B // NW
IPW = BPW * CTX
ICHUNK = 128
NGATHER = IPW // ICHUNK

TV = 1024
NV = (VOCAB + TV - 1) // TV
TAIL = VOCAB - (NV - 1) * TV


@functools.partial(
    pl.kernel,
    out_type=jax.ShapeDtypeStruct((B, D), jnp.float32),
    mesh=plsc.VectorSubcoreMesh(core_axis_name="c", subcore_axis_name="s"),
    compiler_params=pltpu.CompilerParams(use_tc_tiling_on_sc=False),
    scratch_types=[
        pltpu.VMEM((NGATHER, ICHUNK), jnp.int32),
        pltpu.VMEM((IPW, D), jnp.float32),
        pltpu.VMEM((BPW, D), jnp.float32),
        pltpu.SemaphoreType.DMA,
    ],
)
def _sc_gather_pool(ctx_hbm, emb_hbm, ectx_out, idx_v, rows_v, acc_v, sem):
    wid = lax.axis_index("s") * NC + lax.axis_index("c")

    pltpu.sync_copy(ctx_hbm.at[wid], idx_v)

    copies = [
        pltpu.async_copy(emb_hbm.at[idx_v.at[k]],
                         rows_v.at[pl.ds(k * ICHUNK, ICHUNK)], sem)
        for k in range(NGATHER)
    ]
    for c in copies:
        c.wait()

    def pool_row(r, carry):
        base = r * CTX
        for c in range(D // 16):
            acc = rows_v[base, pl.ds(c * 16, 16)]
            for t in range(1, CTX):
                acc = acc + rows_v[base + t, pl.ds(c * 16, 16)]
            acc_v[r, pl.ds(c * 16, 16)] = acc * (1.0 / CTX)
        return carry

    lax.fori_loop(0, BPW, pool_row, 0)

    pltpu.sync_copy(acc_v, ectx_out.at[pl.ds(wid * BPW, BPW)])


def _tc_body(eT_ref, cen_ref, wT_ref, b_ref, out_ref, loss_ref, s_ref, cl_ref):
    j = pl.program_id(0)

    @pl.when(j == 0)
    def _init():
        s_ref[...] = jnp.zeros_like(s_ref)
        cl_ref[...] = jnp.zeros_like(cl_ref)

    logitsT = lax.dot_general(
        wT_ref[...].astype(jnp.bfloat16), eT_ref[...].astype(jnp.bfloat16),
        (((0,), (0,)), ((), ())),
        preferred_element_type=jnp.float32) + b_ref[...]
    out_ref[...] = logitsT

    local = cen_ref[...] - j * TV
    sub = jnp.broadcast_to((local % 8)[:, None, :], (TV // 8, 1, B))
    picked = jnp.take_along_axis(
        logitsT.reshape(TV // 8, 8, B), sub, axis=1)[:, 0, :]
    grp = lax.broadcasted_iota(jnp.int32, (TV // 8, B), 0)
    gmask = grp == (local // 8)
    cl_ref[...] += jnp.sum(jnp.where(gmask, picked, 0.0), axis=0,
                           keepdims=True)

    @pl.when(j < NV - 1)
    def _acc():
        s_ref[...] += jnp.sum(jnp.exp(logitsT), axis=0, keepdims=True)

    @pl.when(j == NV - 1)
    def _tail():
        row2 = lax.broadcasted_iota(jnp.int32, logitsT.shape, 0)
        ex = jnp.where(row2 < TAIL, jnp.exp(logitsT), 0.0)
        s = s_ref[...] + jnp.sum(ex, axis=0, keepdims=True)
        loss_ref[0, 0] = jnp.mean(jnp.log(s) - cl_ref[...])


_tc_project = pl.pallas_call(
    _tc_body,
    grid=(NV,),
    in_specs=[
        pl.BlockSpec((D, B), lambda j: (0, 0)),
        pl.BlockSpec((1, B), lambda j: (0, 0)),
        pl.BlockSpec((D, TV), lambda j: (0, j)),
        pl.BlockSpec((TV, 1), lambda j: (j, 0)),
    ],
    out_specs=[
        pl.BlockSpec((TV, B), lambda j: (j, 0)),
        pl.BlockSpec(memory_space=pltpu.SMEM),
    ],
    out_shape=[
        jax.ShapeDtypeStruct((VOCAB, B), jnp.float32),
        jax.ShapeDtypeStruct((1, 1), jnp.float32),
    ],
    scratch_shapes=[pltpu.VMEM((1, B), jnp.float32),
                    pltpu.VMEM((1, B), jnp.float32)],
    compiler_params=pltpu.CompilerParams(dimension_semantics=("arbitrary",)),
)


def kernel(centers, contexts, emb, W, b):
    centers = centers.astype(jnp.int32)
    ctx3d = contexts.astype(jnp.int32).reshape(NW, NGATHER, ICHUNK)
    e_ctx = _sc_gather_pool(ctx3d, emb)
    logitsT, loss2d = _tc_project(e_ctx.T, centers.reshape(1, B), W.T,
                                  b[:, None])
    return logitsT.T, loss2d[0, 0]
